# baseline (speedup 1.0000x reference)
import functools

import jax
import jax.numpy as jnp
from jax import lax
from jax.experimental import pallas as pl
from jax.experimental.pallas import tpu as pltpu
from jax.experimental.pallas import tpu_sc as plsc

NB = 2
NN = 10000
KK = 16
CI = 128
CO = 128
RR = NB * NN
EPS = 1e-5

NC = 2
NS = 16
NW = NC * NS
RPW = RR // NW
CHUNK = 5
NCHUNK = RPW // CHUNK
GROWS = CHUNK * KK
LANES = 16

MM_BLK = 1000
MM_GRID = RR // MM_BLK


def _mm_body(x_ref, w_ref, o_ref):
    o_ref[...] = lax.dot_general(
        x_ref[...], w_ref[...],
        (((1,), (1,)), ((), ())),
        preferred_element_type=jnp.float32)


def _matmul(x2d, w):
    return pl.pallas_call(
        _mm_body,
        grid=(MM_GRID,),
        in_specs=[
            pl.BlockSpec((MM_BLK, CI), lambda i: (i, 0)),
            pl.BlockSpec((CO, CI), lambda i: (0, 0)),
        ],
        out_specs=pl.BlockSpec((MM_BLK, CO), lambda i: (i, 0)),
        out_shape=jax.ShapeDtypeStruct((RR, CO), jnp.float32),
    )(x2d, w)


def _gm_body(h_hbm, knn_hbm, out_hbm, stats_hbm,
             idx_v, gbuf0, gbuf1, gbuf2, out_v, ssum_v, ssq_v,
             sem0, sem1, sem2):
    cid = lax.axis_index("c")
    sid = lax.axis_index("s")
    wid = sid * NC + cid
    base = wid * RPW

    pltpu.sync_copy(knn_hbm.at[pl.ds(base * KK, RPW * KK)], idx_v)

    off = jnp.full((LANES,), jnp.where(base >= NN, NN, 0), dtype=jnp.int32)

    def _off_body(i, _):
        sl = pl.ds(i * LANES, LANES)
        idx_v[sl] = idx_v[sl] + off
        return 0
    lax.fori_loop(0, RPW * KK // LANES, _off_body, 0)

    zero = jnp.zeros((LANES,), jnp.float32)
    for r in range(CO // LANES):
        ssum_v[pl.ds(r * LANES, LANES)] = zero
        ssq_v[pl.ds(r * LANES, LANES)] = zero

    def _copy(g, buf, sem):
        return pltpu.make_async_copy(
            h_hbm.at[idx_v.at[pl.ds(g * GROWS, GROWS)]], buf, sem)

    def _accum(g, buf):
        for d in range(CHUNK):
            row = g * CHUNK + d
            for r in range(CO // LANES):
                sl = pl.ds(r * LANES, LANES)
                v = [buf[d * KK + j, sl] for j in range(KK)]
                while len(v) > 1:
                    v = [v[2 * t] + v[2 * t + 1] for t in range(len(v) // 2)]
                a = v[0]
                out_v[pl.ds(row * CO + r * LANES, LANES)] = a
                plsc.addupdate(ssum_v.at[sl], a)
                plsc.addupdate(ssq_v.at[sl], a * a)

    slots = ((gbuf0, sem0), (gbuf1, sem1), (gbuf2, sem2))
    nd = len(slots)
    for s, (buf, sem) in enumerate(slots):
        _copy(s, buf, sem).start()

    def _body(i, _):
        for s, (buf, sem) in enumerate(slots):
            g = nd * i + s
            _copy(g, buf, sem).wait()
            _accum(g, buf)

            @pl.when(g + nd < NCHUNK)
            def _():
                _copy(g + nd, buf, sem).start()
        return 0

    lax.fori_loop(0, NCHUNK // nd, _body, 0)

    for s in range(NCHUNK % nd):
        g = (NCHUNK // nd) * nd + s
        buf, sem = slots[s]
        _copy(g, buf, sem).wait()
        _accum(g, buf)

    pltpu.sync_copy(out_v, out_hbm.at[pl.ds(base * CO, RPW * CO)])
    pltpu.sync_copy(ssum_v, stats_hbm.at[pl.ds(wid * CO, CO)])
    pltpu.sync_copy(ssq_v, stats_hbm.at[pl.ds((NW + wid) * CO, CO)])


@functools.lru_cache(maxsize=None)
def _make_gather_mean():
    mesh = plsc.VectorSubcoreMesh(
        core_axis_name="c", subcore_axis_name="s",
        num_cores=NC, num_subcores=NS)
    return pl.kernel(
        _gm_body,
        out_type=(
            jax.ShapeDtypeStruct((RR * CO,), jnp.float32),
            jax.ShapeDtypeStruct((2 * NW * CO,), jnp.float32),
        ),
        mesh=mesh,
        scratch_types=[
            pltpu.VMEM((RPW * KK,), jnp.int32),
            pltpu.VMEM((GROWS, CO), jnp.float32),
            pltpu.VMEM((GROWS, CO), jnp.float32),
            pltpu.VMEM((GROWS, CO), jnp.float32),
            pltpu.VMEM((RPW * CO,), jnp.float32),
            pltpu.VMEM((CO,), jnp.float32),
            pltpu.VMEM((CO,), jnp.float32),
            pltpu.SemaphoreType.DMA,
            pltpu.SemaphoreType.DMA,
            pltpu.SemaphoreType.DMA,
        ],
    )


def _bn_body(s_ref, st_ref, g_ref, b_ref, o_ref):
    st = st_ref[...]
    s1 = jnp.sum(st[:NW], axis=0, keepdims=True)
    s2 = jnp.sum(st[NW:], axis=0, keepdims=True)
    mean = s1 / RR
    var = s2 / RR - mean * mean
    alpha = g_ref[...] * lax.rsqrt(var + (KK * KK) * EPS)
    shift = b_ref[...] - mean * alpha
    o_ref[...] = s_ref[...] * alpha + shift


def _bn(sums, stats2d, gamma2d, beta2d):
    return pl.pallas_call(
        _bn_body,
        grid=(MM_GRID,),
        in_specs=[
            pl.BlockSpec((MM_BLK, CO), lambda i: (i, 0)),
            pl.BlockSpec((2 * NW, CO), lambda i: (0, 0)),
            pl.BlockSpec((1, CO), lambda i: (0, 0)),
            pl.BlockSpec((1, CO), lambda i: (0, 0)),
        ],
        out_specs=pl.BlockSpec((MM_BLK, CO), lambda i: (i, 0)),
        out_shape=jax.ShapeDtypeStruct((RR, CO), jnp.float32),
    )(sums, stats2d, gamma2d, beta2d)


@jax.jit
def kernel(x, knn, W, gamma, beta):
    h = _matmul(x.reshape(RR, CI), W)
    sums, stats = _make_gather_mean()(h, knn.reshape(RR * KK))
    out = _bn(sums.reshape(RR, CO), stats.reshape(2 * NW, CO),
              gamma.reshape(1, CO), beta.reshape(1, CO))
    return out.reshape(NB, NN, CO)


# speedup vs baseline: 1.0848x; 1.0848x over previous
import functools

import jax
import jax.numpy as jnp
from jax import lax
from jax.experimental import pallas as pl
from jax.experimental.pallas import tpu as pltpu
from jax.experimental.pallas import tpu_sc as plsc

NB = 2
NN = 10000
KK = 16
CI = 128
CO = 128
RR = NB * NN
EPS = 1e-5

NC = 2
NS = 16
NW = NC * NS
RPW = RR // NW
CHUNK = 5
NCHUNK = RPW // CHUNK
GROWS = CHUNK * KK
LANES = 16

MM_BLK = 1000
MM_GRID = RR // MM_BLK


def _mm_body(x_ref, w_ref, o_ref):
    o_ref[...] = lax.dot_general(
        x_ref[...], w_ref[...],
        (((1,), (1,)), ((), ())),
        preferred_element_type=jnp.float32)


def _matmul(x2d, w):
    return pl.pallas_call(
        _mm_body,
        grid=(MM_GRID,),
        in_specs=[
            pl.BlockSpec((MM_BLK, CI), lambda i: (i, 0)),
            pl.BlockSpec((CO, CI), lambda i: (0, 0)),
        ],
        out_specs=pl.BlockSpec((MM_BLK, CO), lambda i: (i, 0)),
        out_shape=jax.ShapeDtypeStruct((RR, CO), jnp.float32),
    )(x2d, w)


def _gm_body(h_hbm, knn_hbm, out_hbm, stats_hbm,
             h_sh, idx_v, gbuf0, gbuf1, ob0, ob1, ssum_v, ssq_v,
             gsem0, gsem1, osem0, osem1):
    cid = lax.axis_index("c")
    sid = lax.axis_index("s")
    base = cid * NN + sid * RPW

    @pl.when(sid == 0)
    def _():
        pltpu.sync_copy(h_hbm.at[pl.ds(cid * NN, NN)], h_sh)

    pltpu.sync_copy(knn_hbm.at[pl.ds(base * KK, RPW * KK)], idx_v)
    plsc.subcore_barrier()

    zero = jnp.zeros((LANES,), jnp.float32)
    for r in range(CO // LANES):
        ssum_v[pl.ds(r * LANES, LANES)] = zero
        ssq_v[pl.ds(r * LANES, LANES)] = zero

    def _gather(g, buf, sem):
        return pltpu.make_async_copy(
            h_sh.at[idx_v.at[pl.ds(g * GROWS, GROWS)]], buf, sem)

    def _outdma(g, ob, sem):
        return pltpu.make_async_copy(
            ob, out_hbm.at[pl.ds((base + g * CHUNK) * CO, CHUNK * CO)], sem)

    def _accum(buf, ob):
        for d in range(CHUNK):
            for r in range(CO // LANES):
                sl = pl.ds(r * LANES, LANES)
                v = [buf[d * KK + j, sl] for j in range(KK)]
                while len(v) > 1:
                    v = [v[2 * t] + v[2 * t + 1] for t in range(len(v) // 2)]
                a = v[0]
                ob[pl.ds(d * CO + r * LANES, LANES)] = a
                plsc.addupdate(ssum_v.at[sl], a)
                plsc.addupdate(ssq_v.at[sl], a * a)

    slots = ((gbuf0, gsem0, ob0, osem0), (gbuf1, gsem1, ob1, osem1))
    _gather(0, gbuf0, gsem0).start()
    _gather(1, gbuf1, gsem1).start()

    def _body(i, _):
        for s, (gb, gs, ob, osem) in enumerate(slots):
            g = 2 * i + s
            _gather(g, gb, gs).wait()

            @pl.when(g >= 2)
            def _():
                _outdma(g - 2, ob, osem).wait()

            _accum(gb, ob)
            _outdma(g, ob, osem).start()

            @pl.when(g + 2 < NCHUNK)
            def _():
                _gather(g + 2, gb, gs).start()
        return 0

    lax.fori_loop(0, (NCHUNK - 1) // 2, _body, 0)

    g_last = NCHUNK - 1
    _gather(g_last, gbuf0, gsem0).wait()
    _outdma(g_last - 2, ob0, osem0).wait()
    _accum(gbuf0, ob0)
    _outdma(g_last, ob0, osem0).start()
    _outdma(g_last - 1, ob1, osem1).wait()
    _outdma(g_last, ob0, osem0).wait()

    wid = cid * NS + sid
    pltpu.sync_copy(ssum_v, stats_hbm.at[pl.ds(wid * CO, CO)])
    pltpu.sync_copy(ssq_v, stats_hbm.at[pl.ds((NW + wid) * CO, CO)])


@functools.lru_cache(maxsize=None)
def _make_gather_mean():
    mesh = plsc.VectorSubcoreMesh(
        core_axis_name="c", subcore_axis_name="s",
        num_cores=NC, num_subcores=NS)
    return pl.kernel(
        _gm_body,
        out_type=(
            jax.ShapeDtypeStruct((RR * CO,), jnp.float32),
            jax.ShapeDtypeStruct((2 * NW * CO,), jnp.float32),
        ),
        mesh=mesh,
        scratch_types=[
            pltpu.VMEM_SHARED((NN, CO), jnp.float32),
            pltpu.VMEM((RPW * KK,), jnp.int32),
            pltpu.VMEM((GROWS, CO), jnp.float32),
            pltpu.VMEM((GROWS, CO), jnp.float32),
            pltpu.VMEM((CHUNK * CO,), jnp.float32),
            pltpu.VMEM((CHUNK * CO,), jnp.float32),
            pltpu.VMEM((CO,), jnp.float32),
            pltpu.VMEM((CO,), jnp.float32),
            pltpu.SemaphoreType.DMA,
            pltpu.SemaphoreType.DMA,
            pltpu.SemaphoreType.DMA,
            pltpu.SemaphoreType.DMA,
        ],
    )


def _bn_body(s_ref, st_ref, g_ref, b_ref, o_ref):
    st = st_ref[...]
    s1 = jnp.sum(st[:NW], axis=0, keepdims=True)
    s2 = jnp.sum(st[NW:], axis=0, keepdims=True)
    mean = s1 / RR
    var = s2 / RR - mean * mean
    alpha = g_ref[...] * lax.rsqrt(var + (KK * KK) * EPS)
    shift = b_ref[...] - mean * alpha
    o_ref[...] = s_ref[...] * alpha + shift


def _bn(sums, stats2d, gamma2d, beta2d):
    return pl.pallas_call(
        _bn_body,
        grid=(MM_GRID,),
        in_specs=[
            pl.BlockSpec((MM_BLK, CO), lambda i: (i, 0)),
            pl.BlockSpec((2 * NW, CO), lambda i: (0, 0)),
            pl.BlockSpec((1, CO), lambda i: (0, 0)),
            pl.BlockSpec((1, CO), lambda i: (0, 0)),
        ],
        out_specs=pl.BlockSpec((MM_BLK, CO), lambda i: (i, 0)),
        out_shape=jax.ShapeDtypeStruct((RR, CO), jnp.float32),
    )(sums, stats2d, gamma2d, beta2d)


@jax.jit
def kernel(x, knn, W, gamma, beta):
    h = _matmul(x.reshape(RR, CI), W)
    sums, stats = _make_gather_mean()(h, knn.reshape(RR * KK))
    out = _bn(sums.reshape(RR, CO), stats.reshape(2 * NW, CO),
              gamma.reshape(1, CO), beta.reshape(1, CO))
    return out.reshape(NB, NN, CO)


# speedup vs baseline: 1.4678x; 1.3531x over previous
import functools

import jax
import jax.numpy as jnp
from jax import lax
from jax.experimental import pallas as pl
from jax.experimental.pallas import tpu as pltpu
from jax.experimental.pallas import tpu_sc as plsc

NB = 2
NN = 10000
KK = 16
CI = 128
CO = 128
RR = NB * NN
EPS = 1e-5

NC = 2
NS = 16
NW = NC * NS
RPW = RR // NW
CHUNK = 5
NCHUNK = RPW // CHUNK
GROWS = CHUNK * KK
LANES = 16

MM_BLK = 1000
MM_GRID = RR // MM_BLK


def _mm_body(x_ref, w_ref, o_ref):
    o_ref[...] = lax.dot_general(
        x_ref[...], w_ref[...],
        (((1,), (1,)), ((), ())),
        preferred_element_type=jnp.float32)


def _matmul(x2d, w):
    return pl.pallas_call(
        _mm_body,
        grid=(MM_GRID,),
        in_specs=[
            pl.BlockSpec((MM_BLK, CI), lambda i: (i, 0)),
            pl.BlockSpec((CO, CI), lambda i: (0, 0)),
        ],
        out_specs=pl.BlockSpec((MM_BLK, CO), lambda i: (i, 0)),
        out_shape=jax.ShapeDtypeStruct((RR, CO), jnp.float32),
    )(x2d, w)


ZROWS = 640


def _gm_body(h_hbm, knn_hbm, out_hbm, stats_hbm,
             acc_sh, idx_v, dstidx, ridx, gbuf0, gbuf1, ssum_v, ssq_v,
             gsem0, gsem1, ssem0, ssem1, osem):
    cid = lax.axis_index("c")
    sid = lax.axis_index("s")
    wid = cid * NS + sid
    base = wid * RPW
    sbase = sid * RPW

    pltpu.sync_copy(knn_hbm.at[pl.ds(base * KK, RPW * KK)], idx_v)

    zerof = jnp.zeros((LANES,), jnp.float32)
    for j in range(GROWS):
        for r in range(CO // LANES):
            gbuf0[j, pl.ds(r * LANES, LANES)] = zerof
    nfull = lax.select(sid < NS - 1, ZROWS // GROWS,
                       (NS * RPW - (NS - 1) * ZROWS) // GROWS)

    def _z_body(q, _):
        pltpu.sync_copy(gbuf0, acc_sh.at[pl.ds(sid * ZROWS + q * GROWS,
                                               GROWS)])
        return 0
    lax.fori_loop(0, nfull, _z_body, 0)

    for r in range(CO // LANES):
        ssum_v[pl.ds(r * LANES, LANES)] = zerof
        ssq_v[pl.ds(r * LANES, LANES)] = zerof

    off = jnp.full((LANES,), cid * NN, dtype=jnp.int32)

    def _off_body(i, _):
        sl = pl.ds(i * LANES, LANES)
        idx_v[sl] = idx_v[sl] + off
        return 0
    lax.fori_loop(0, RPW * KK // LANES, _off_body, 0)

    def _di_body(g, _):
        for d in range(CHUNK):
            dstidx[g, pl.ds(d * LANES, LANES)] = jnp.full(
                (LANES,), sbase + g * CHUNK + d, jnp.int32)
        return 0
    lax.fori_loop(0, NCHUNK, _di_body, 0)

    lane = lax.iota(jnp.int32, LANES)

    def _ri_body(i, _):
        ridx[pl.ds(i * LANES, LANES)] = sbase + jnp.minimum(
            i * LANES + lane, RPW - 1)
        return 0
    lax.fori_loop(0, (RPW + LANES) // LANES, _ri_body, 0)

    plsc.subcore_barrier()

    def _gather(g, buf, sem):
        return pltpu.make_async_copy(
            h_hbm.at[idx_v.at[pl.ds(g * GROWS, GROWS)]], buf, sem)

    def _scat_start(g, buf, sem):
        pltpu.async_copy(buf, acc_sh.at[dstidx.at[g]], sem, add=True)

    def _scat_wait(g, buf, sem):
        pltpu.make_async_copy(buf, acc_sh.at[dstidx.at[g]], sem).wait()

    slots = ((gbuf0, gsem0, ssem0), (gbuf1, gsem1, ssem1))
    _gather(0, gbuf0, gsem0).start()
    _gather(1, gbuf1, gsem1).start()

    def _body(i, _):
        for s, (gb, gs, ss) in enumerate(slots):
            g = 2 * i + s
            _gather(g, gb, gs).wait()
            _scat_start(g, gb, ss)
            _scat_wait(g, gb, ss)

            @pl.when(g + 2 < NCHUNK)
            def _():
                _gather(g + 2, gb, gs).start()
        return 0

    lax.fori_loop(0, (NCHUNK - 1) // 2, _body, 0)

    g_last = NCHUNK - 1
    _gather(g_last, gbuf0, gsem0).wait()
    _scat_start(g_last, gbuf0, ssem0)
    _scat_wait(g_last, gbuf0, ssem0)

    plsc.subcore_barrier()

    @pl.when(sid == 0)
    def _():
        pltpu.async_copy(acc_sh, out_hbm.at[cid], osem)

    NRB = (RPW + GROWS) // GROWS

    def _rb(q, buf, sem):
        return pltpu.make_async_copy(
            acc_sh.at[ridx.at[pl.ds(q * GROWS, GROWS)]], buf, sem)

    _rb(0, gbuf0, gsem0).start()
    for q in range(NRB):
        gb, gs, _ = slots[q % 2]
        _rb(q, gb, gs).wait()
        if q + 1 < NRB:
            nb, ns, _ = slots[(q + 1) % 2]
            _rb(q + 1, nb, ns).start()
        nrows = GROWS if (q + 1) * GROWS <= RPW else RPW - q * GROWS

        def _srow(j, _):
            for r in range(CO // LANES):
                sl = pl.ds(r * LANES, LANES)
                a = gb[j, sl]
                plsc.addupdate(ssum_v.at[sl], a)
                plsc.addupdate(ssq_v.at[sl], a * a)
            return 0
        lax.fori_loop(0, nrows, _srow, 0)

    pltpu.sync_copy(ssum_v, stats_hbm.at[pl.ds(wid * CO, CO)])
    pltpu.sync_copy(ssq_v, stats_hbm.at[pl.ds((NW + wid) * CO, CO)])

    @pl.when(sid == 0)
    def _():
        pltpu.make_async_copy(acc_sh, out_hbm.at[cid], osem).wait()


@functools.lru_cache(maxsize=None)
def _make_gather_mean():
    mesh = plsc.VectorSubcoreMesh(
        core_axis_name="c", subcore_axis_name="s",
        num_cores=NC, num_subcores=NS)
    return pl.kernel(
        _gm_body,
        out_type=(
            jax.ShapeDtypeStruct((NC, NS * RPW, CO), jnp.float32),
            jax.ShapeDtypeStruct((2 * NW * CO,), jnp.float32),
        ),
        mesh=mesh,
        scratch_types=[
            pltpu.VMEM_SHARED((NS * RPW, CO), jnp.float32),
            pltpu.VMEM((RPW * KK,), jnp.int32),
            pltpu.VMEM((NCHUNK, GROWS), jnp.int32),
            pltpu.VMEM((ZROWS,), jnp.int32),
            pltpu.VMEM((GROWS, CO), jnp.float32),
            pltpu.VMEM((GROWS, CO), jnp.float32),
            pltpu.VMEM((CO,), jnp.float32),
            pltpu.VMEM((CO,), jnp.float32),
            pltpu.SemaphoreType.DMA,
            pltpu.SemaphoreType.DMA,
            pltpu.SemaphoreType.DMA,
            pltpu.SemaphoreType.DMA,
            pltpu.SemaphoreType.DMA,
        ],
    )


def _bn_body(s_ref, st_ref, g_ref, b_ref, o_ref):
    st = st_ref[...]
    s1 = jnp.sum(st[:NW], axis=0, keepdims=True)
    s2 = jnp.sum(st[NW:], axis=0, keepdims=True)
    mean = s1 / RR
    var = s2 / RR - mean * mean
    alpha = g_ref[...] * lax.rsqrt(var + (KK * KK) * EPS)
    shift = b_ref[...] - mean * alpha
    o_ref[...] = s_ref[...] * alpha + shift


def _bn(sums, stats2d, gamma2d, beta2d):
    return pl.pallas_call(
        _bn_body,
        grid=(MM_GRID,),
        in_specs=[
            pl.BlockSpec((MM_BLK, CO), lambda i: (i, 0)),
            pl.BlockSpec((2 * NW, CO), lambda i: (0, 0)),
            pl.BlockSpec((1, CO), lambda i: (0, 0)),
            pl.BlockSpec((1, CO), lambda i: (0, 0)),
        ],
        out_specs=pl.BlockSpec((MM_BLK, CO), lambda i: (i, 0)),
        out_shape=jax.ShapeDtypeStruct((RR, CO), jnp.float32),
    )(sums, stats2d, gamma2d, beta2d)


@jax.jit
def kernel(x, knn, W, gamma, beta):
    h = _matmul(x.reshape(RR, CI), W)
    sums3, stats = _make_gather_mean()(h, knn.reshape(RR * KK))
    out = _bn(sums3.reshape(RR, CO), stats.reshape(2 * NW, CO),
              gamma.reshape(1, CO), beta.reshape(1, CO))
    return out.reshape(NB, NN, CO)


# speedup vs baseline: 1.4764x; 1.0058x over previous
import functools

import jax
import jax.numpy as jnp
from jax import lax
from jax.experimental import pallas as pl
from jax.experimental.pallas import tpu as pltpu
from jax.experimental.pallas import tpu_sc as plsc

NB = 2
NN = 10000
KK = 16
CI = 128
CO = 128
RR = NB * NN
EPS = 1e-5

NC = 2
NS = 16
NW = NC * NS
RPW = RR // NW
CHUNK = 5
GROWS = CHUNK * KK
NCHUNK = (RPW + CHUNK - 1) // CHUNK
NIDX = NCHUNK * GROWS
SPARE = NS * RPW
LANES = 16
ZCOPY = 80
RBROWS = GROWS
NRB = (RPW + RBROWS - 1) // RBROWS

MM_BLK = 1000
MM_GRID = RR // MM_BLK


def _mm_body(x_ref, w_ref, o_ref):
    o_ref[...] = lax.dot_general(
        x_ref[...], w_ref[...],
        (((1,), (1,)), ((), ())),
        preferred_element_type=jnp.float32)


def _matmul(x2d, w):
    return pl.pallas_call(
        _mm_body,
        grid=(MM_GRID,),
        in_specs=[
            pl.BlockSpec((MM_BLK, CI), lambda i: (i, 0)),
            pl.BlockSpec((CO, CI), lambda i: (0, 0)),
        ],
        out_specs=pl.BlockSpec((MM_BLK, CO), lambda i: (i, 0)),
        out_shape=jax.ShapeDtypeStruct((RR, CO), jnp.float32),
    )(x2d, w)


ZROWS = 640


def _gm_body(h_hbm, knn_hbm, out_hbm, stats_hbm,
             acc_sh, idx_v, dstidx, ridx, ridx2, gbuf0, gbuf1,
             ssum_v, ssq_v, gsem0, gsem1, ssem0, ssem1, osem):
    cid = lax.axis_index("c")
    sid = lax.axis_index("s")
    wid = cid * NS + sid
    base = wid * RPW
    sbase = sid * RPW

    pltpu.sync_copy(knn_hbm.at[pl.ds(base * KK, RPW * KK)],
                    idx_v.at[pl.ds(0, RPW * KK)])
    zeroi = jnp.zeros((LANES,), jnp.int32)
    for i in range((NIDX - RPW * KK) // LANES):
        idx_v[pl.ds(RPW * KK + i * LANES, LANES)] = zeroi

    zerof = jnp.zeros((LANES,), jnp.float32)

    def _zg_body(j, _):
        for r in range(CO // LANES):
            gbuf0[j, pl.ds(r * LANES, LANES)] = zerof
        return 0
    lax.fori_loop(0, ZCOPY, _zg_body, 0)
    nfull = lax.select(sid < NS - 1, ZROWS // ZCOPY,
                       (NS * RPW - (NS - 1) * ZROWS) // ZCOPY)

    def _z_body(q, _):
        pltpu.async_copy(gbuf0.at[pl.ds(0, ZCOPY)],
                         acc_sh.at[pl.ds(sid * ZROWS + q * ZCOPY, ZCOPY)],
                         osem)
        return 0
    lax.fori_loop(0, nfull, _z_body, 0)

    for r in range(CO // LANES):
        ssum_v[pl.ds(r * LANES, LANES)] = zerof
        ssq_v[pl.ds(r * LANES, LANES)] = zerof

    off = jnp.full((LANES,), cid * NN, dtype=jnp.int32)

    def _off_body(i, _):
        sl = pl.ds(i * LANES, LANES)
        idx_v[sl] = idx_v[sl] + off
        return 0
    lax.fori_loop(0, NIDX // LANES, _off_body, 0)

    def _di_body(g, _):
        for d in range(CHUNK):
            row = g * CHUNK + d
            val = lax.select(row <= RPW - 1, sbase + row, SPARE)
            dstidx[g, pl.ds(d * LANES, LANES)] = jnp.full(
                (LANES,), val, jnp.int32)
        return 0
    lax.fori_loop(0, NCHUNK, _di_body, 0)

    lane = lax.iota(jnp.int32, LANES)

    def _ri_body(i, _):
        ridx[pl.ds(i * LANES, LANES)] = sbase + jnp.minimum(
            i * LANES + lane, RPW - 1)
        return 0
    lax.fori_loop(0, NRB * RBROWS // LANES, _ri_body, 0)

    def _r2_body(q, _):
        for t in range(RBROWS // LANES):
            ridx2[q, pl.ds(t * LANES, LANES)] = base + jnp.minimum(
                q * RBROWS + t * LANES + lane, RPW - 1)
        return 0
    lax.fori_loop(0, NRB, _r2_body, 0)

    def _zw_body(q, _):
        pltpu.make_async_copy(
            gbuf0.at[pl.ds(0, ZCOPY)],
            acc_sh.at[pl.ds(sid * ZROWS + q * ZCOPY, ZCOPY)], osem).wait()
        return 0
    lax.fori_loop(0, nfull, _zw_body, 0)
    plsc.subcore_barrier()

    def _gather(g, buf, sem):
        return pltpu.make_async_copy(
            h_hbm.at[idx_v.at[pl.ds(g * GROWS, GROWS)]], buf, sem)

    def _scat_start(g, buf, sem):
        pltpu.async_copy(buf, acc_sh.at[dstidx.at[g]], sem, add=True)

    def _scat_wait(g, buf, sem):
        pltpu.make_async_copy(buf, acc_sh.at[dstidx.at[g]], sem).wait()

    slots = ((gbuf0, gsem0, ssem0), (gbuf1, gsem1, ssem1))
    _gather(0, gbuf0, gsem0).start()
    _gather(1, gbuf1, gsem1).start()

    def _body(i, _):
        for s, (gb, gs, ss) in enumerate(slots):
            g = 2 * i + s
            _gather(g, gb, gs).wait()
            _scat_start(g, gb, ss)
            _scat_wait(g, gb, ss)

            @pl.when(g + 2 < NCHUNK)
            def _():
                _gather(g + 2, gb, gs).start()
        return 0

    lax.fori_loop(0, (NCHUNK - 1) // 2, _body, 0)

    g_last = NCHUNK - 1
    _gather(g_last, gbuf0, gsem0).wait()
    _scat_start(g_last, gbuf0, ssem0)
    _scat_wait(g_last, gbuf0, ssem0)

    def _rb(q, buf, sem):
        return pltpu.make_async_copy(
            acc_sh.at[ridx.at[pl.ds(q * RBROWS, RBROWS)]], buf, sem)

    def _oscat(q, buf):
        return pltpu.make_async_copy(buf, out_hbm.at[ridx2.at[q]], osem)

    _rb(0, gbuf0, gsem0).start()
    _rb(1, gbuf1, gsem1).start()
    for q in range(NRB):
        gb, gs, _ = slots[q % 2]
        _rb(q, gb, gs).wait()
        _oscat(q, gb).start()
        nrows = RBROWS if (q + 1) * RBROWS <= RPW else RPW - q * RBROWS

        def _srow(j, _):
            for r in range(CO // LANES):
                sl = pl.ds(r * LANES, LANES)
                a = gb[j, sl]
                plsc.addupdate(ssum_v.at[sl], a)
                plsc.addupdate(ssq_v.at[sl], a * a)
            return 0
        lax.fori_loop(0, nrows, _srow, 0)
        _oscat(q, gb).wait()
        if q + 2 < NRB:
            _rb(q + 2, gb, gs).start()

    pltpu.sync_copy(ssum_v, stats_hbm.at[pl.ds(wid * CO, CO)])
    pltpu.sync_copy(ssq_v, stats_hbm.at[pl.ds((NW + wid) * CO, CO)])


@functools.lru_cache(maxsize=None)
def _make_gather_mean():
    mesh = plsc.VectorSubcoreMesh(
        core_axis_name="c", subcore_axis_name="s",
        num_cores=NC, num_subcores=NS)
    return pl.kernel(
        _gm_body,
        out_type=(
            jax.ShapeDtypeStruct((RR, CO), jnp.float32),
            jax.ShapeDtypeStruct((2 * NW * CO,), jnp.float32),
        ),
        mesh=mesh,
        scratch_types=[
            pltpu.VMEM_SHARED((NS * RPW + 8, CO), jnp.float32),
            pltpu.VMEM((NIDX,), jnp.int32),
            pltpu.VMEM((NCHUNK, GROWS), jnp.int32),
            pltpu.VMEM((NRB * RBROWS,), jnp.int32),
            pltpu.VMEM((NRB, RBROWS), jnp.int32),
            pltpu.VMEM((GROWS, CO), jnp.float32),
            pltpu.VMEM((GROWS, CO), jnp.float32),
            pltpu.VMEM((CO,), jnp.float32),
            pltpu.VMEM((CO,), jnp.float32),
            pltpu.SemaphoreType.DMA,
            pltpu.SemaphoreType.DMA,
            pltpu.SemaphoreType.DMA,
            pltpu.SemaphoreType.DMA,
            pltpu.SemaphoreType.DMA,
        ],
    )


def _bn_body(s_ref, st_ref, g_ref, b_ref, o_ref):
    st = st_ref[...]
    s1 = jnp.sum(st[:NW], axis=0, keepdims=True)
    s2 = jnp.sum(st[NW:], axis=0, keepdims=True)
    mean = s1 / RR
    var = s2 / RR - mean * mean
    alpha = g_ref[...] * lax.rsqrt(var + (KK * KK) * EPS)
    shift = b_ref[...] - mean * alpha
    o_ref[...] = s_ref[...] * alpha + shift


def _bn(sums, stats2d, gamma2d, beta2d):
    return pl.pallas_call(
        _bn_body,
        grid=(MM_GRID,),
        in_specs=[
            pl.BlockSpec((MM_BLK, CO), lambda i: (i, 0)),
            pl.BlockSpec((2 * NW, CO), lambda i: (0, 0)),
            pl.BlockSpec((1, CO), lambda i: (0, 0)),
            pl.BlockSpec((1, CO), lambda i: (0, 0)),
        ],
        out_specs=pl.BlockSpec((MM_BLK, CO), lambda i: (i, 0)),
        out_shape=jax.ShapeDtypeStruct((RR, CO), jnp.float32),
    )(sums, stats2d, gamma2d, beta2d)


@jax.jit
def kernel(x, knn, W, gamma, beta):
    h = _matmul(x.reshape(RR, CI), W)
    sums, stats = _make_gather_mean()(h, knn.reshape(RR * KK))
    out = _bn(sums, stats.reshape(2 * NW, CO),
              gamma.reshape(1, CO), beta.reshape(1, CO))
    return out.reshape(NB, NN, CO)


# speedup vs baseline: 1.4820x; 1.0038x over previous
import functools

import jax
import jax.numpy as jnp
from jax import lax
from jax.experimental import pallas as pl
from jax.experimental.pallas import tpu as pltpu
from jax.experimental.pallas import tpu_sc as plsc

NB = 2
NN = 10000
KK = 16
CI = 128
CO = 128
RR = NB * NN
EPS = 1e-5

NC = 2
NS = 16
NW = NC * NS
RPW = RR // NW
CHUNK = 5
GROWS = CHUNK * KK
NCHUNK = (RPW + CHUNK - 1) // CHUNK
NIDX = NCHUNK * GROWS
SPARE = NS * RPW
LANES = 16
ZCOPY = 80
RBROWS = GROWS
NRB = (RPW + RBROWS - 1) // RBROWS

MM_BLK = 1000
MM_GRID = RR // MM_BLK


def _mm_body(x_ref, w_ref, o_ref):
    o_ref[...] = lax.dot_general(
        x_ref[...], w_ref[...],
        (((1,), (1,)), ((), ())),
        preferred_element_type=jnp.float32)


def _matmul(x2d, w):
    return pl.pallas_call(
        _mm_body,
        grid=(MM_GRID,),
        in_specs=[
            pl.BlockSpec((MM_BLK, CI), lambda i: (i, 0)),
            pl.BlockSpec((CO, CI), lambda i: (0, 0)),
        ],
        out_specs=pl.BlockSpec((MM_BLK, CO), lambda i: (i, 0)),
        out_shape=jax.ShapeDtypeStruct((RR, CO), jnp.float32),
    )(x2d, w)


ZROWS = 640


def _gm_body(h_hbm, knn_hbm, out_hbm, stats_hbm,
             acc_sh, idx_v, dstidx, ridx, ridx2, gbuf0, gbuf1,
             ssum_v, ssq_v, gsem0, gsem1, ssem0, ssem1, osem):
    cid = lax.axis_index("c")
    sid = lax.axis_index("s")
    wid = cid * NS + sid
    base = wid * RPW
    sbase = sid * RPW

    pltpu.sync_copy(knn_hbm.at[pl.ds(base * KK, RPW * KK)],
                    idx_v.at[pl.ds(0, RPW * KK)])
    zeroi = jnp.zeros((LANES,), jnp.int32)
    for i in range((NIDX - RPW * KK) // LANES):
        idx_v[pl.ds(RPW * KK + i * LANES, LANES)] = zeroi

    zerof = jnp.zeros((LANES,), jnp.float32)

    def _zg_body(j, _):
        for r in range(CO // LANES):
            gbuf0[j, pl.ds(r * LANES, LANES)] = zerof
        return 0
    lax.fori_loop(0, ZCOPY, _zg_body, 0)
    nfull = lax.select(sid < NS - 1, ZROWS // ZCOPY,
                       (NS * RPW - (NS - 1) * ZROWS) // ZCOPY)

    def _z_body(q, _):
        pltpu.async_copy(gbuf0.at[pl.ds(0, ZCOPY)],
                         acc_sh.at[pl.ds(sid * ZROWS + q * ZCOPY, ZCOPY)],
                         osem)
        return 0
    lax.fori_loop(0, nfull, _z_body, 0)

    for r in range(CO // LANES):
        ssum_v[pl.ds(r * LANES, LANES)] = zerof
        ssq_v[pl.ds(r * LANES, LANES)] = zerof

    off = jnp.full((LANES,), cid * NN, dtype=jnp.int32)

    def _off_body(i, _):
        sl = pl.ds(i * LANES, LANES)
        idx_v[sl] = idx_v[sl] + off
        return 0
    lax.fori_loop(0, NIDX // LANES, _off_body, 0)

    def _di_body(g, _):
        for d in range(CHUNK):
            row = g * CHUNK + d
            val = lax.select(row <= RPW - 1, sbase + row, SPARE)
            dstidx[g, pl.ds(d * LANES, LANES)] = jnp.full(
                (LANES,), val, jnp.int32)
        return 0
    lax.fori_loop(0, NCHUNK, _di_body, 0)

    lane = lax.iota(jnp.int32, LANES)

    def _ri_body(i, _):
        ridx[pl.ds(i * LANES, LANES)] = sbase + jnp.minimum(
            i * LANES + lane, RPW - 1)
        return 0
    lax.fori_loop(0, NRB * RBROWS // LANES, _ri_body, 0)

    def _r2_body(q, _):
        for t in range(RBROWS // LANES):
            ridx2[q, pl.ds(t * LANES, LANES)] = base + jnp.minimum(
                q * RBROWS + t * LANES + lane, RPW - 1)
        return 0
    lax.fori_loop(0, NRB, _r2_body, 0)

    def _zw_body(q, _):
        pltpu.make_async_copy(
            gbuf0.at[pl.ds(0, ZCOPY)],
            acc_sh.at[pl.ds(sid * ZROWS + q * ZCOPY, ZCOPY)], osem).wait()
        return 0
    lax.fori_loop(0, nfull, _zw_body, 0)
    plsc.subcore_barrier()

    def _gather(g, buf, sem):
        return pltpu.make_async_copy(
            h_hbm.at[idx_v.at[pl.ds(g * GROWS, GROWS)]], buf, sem)

    def _scat_start(g, buf, sem):
        pltpu.async_copy(buf, acc_sh.at[dstidx.at[g]], sem, add=True)

    def _scat_wait(g, buf, sem):
        pltpu.make_async_copy(buf, acc_sh.at[dstidx.at[g]], sem).wait()

    slots = ((gbuf0, gsem0, ssem0), (gbuf1, gsem1, ssem1))
    _gather(0, gbuf0, gsem0).start()
    _gather(1, gbuf1, gsem1).start()

    def _body(i, _):
        for s, (gb, gs, ss) in enumerate(slots):
            g = 2 * i + s
            _gather(g, gb, gs).wait()
            _scat_start(g, gb, ss)
            _scat_wait(g, gb, ss)

            @pl.when(g + 2 < NCHUNK)
            def _():
                _gather(g + 2, gb, gs).start()
        return 0

    lax.fori_loop(0, (NCHUNK - 1) // 2, _body, 0)

    g_last = NCHUNK - 1
    _gather(g_last, gbuf0, gsem0).wait()
    _scat_start(g_last, gbuf0, ssem0)
    _scat_wait(g_last, gbuf0, ssem0)

    def _rb(q, buf, sem):
        return pltpu.make_async_copy(
            acc_sh.at[ridx.at[pl.ds(q * RBROWS, RBROWS)]], buf, sem)

    def _oscat(q, buf):
        return pltpu.make_async_copy(buf, out_hbm.at[ridx2.at[q]], osem)

    _rb(0, gbuf0, gsem0).start()
    _rb(1, gbuf1, gsem1).start()
    for q in range(NRB):
        gb, gs, _ = slots[q % 2]
        _rb(q, gb, gs).wait()
        _oscat(q, gb).start()
        nrows = RBROWS if (q + 1) * RBROWS <= RPW else RPW - q * RBROWS

        def _srow(j, _):
            for r in range(CO // LANES):
                sl = pl.ds(r * LANES, LANES)
                a = gb[j, sl]
                plsc.addupdate(ssum_v.at[sl], a)
                plsc.addupdate(ssq_v.at[sl], a * a)
            return 0
        lax.fori_loop(0, nrows, _srow, 0)
        _oscat(q, gb).wait()
        if q + 2 < NRB:
            _rb(q + 2, gb, gs).start()

    pltpu.sync_copy(ssum_v, stats_hbm.at[pl.ds(wid * CO, CO)])
    pltpu.sync_copy(ssq_v, stats_hbm.at[pl.ds((NW + wid) * CO, CO)])


@functools.lru_cache(maxsize=None)
def _make_gather_mean():
    mesh = plsc.VectorSubcoreMesh(
        core_axis_name="c", subcore_axis_name="s",
        num_cores=NC, num_subcores=NS)
    return pl.kernel(
        _gm_body,
        out_type=(
            jax.ShapeDtypeStruct((RR, CO), jnp.float32),
            jax.ShapeDtypeStruct((2 * NW * CO,), jnp.float32),
        ),
        mesh=mesh,
        scratch_types=[
            pltpu.VMEM_SHARED((NS * RPW + 8, CO), jnp.float32),
            pltpu.VMEM((NIDX,), jnp.int32),
            pltpu.VMEM((NCHUNK, GROWS), jnp.int32),
            pltpu.VMEM((NRB * RBROWS,), jnp.int32),
            pltpu.VMEM((NRB, RBROWS), jnp.int32),
            pltpu.VMEM((GROWS, CO), jnp.float32),
            pltpu.VMEM((GROWS, CO), jnp.float32),
            pltpu.VMEM((CO,), jnp.float32),
            pltpu.VMEM((CO,), jnp.float32),
            pltpu.SemaphoreType.DMA,
            pltpu.SemaphoreType.DMA,
            pltpu.SemaphoreType.DMA,
            pltpu.SemaphoreType.DMA,
            pltpu.SemaphoreType.DMA,
        ],
        compiler_params=pltpu.CompilerParams(use_tc_tiling_on_sc=False),
    )


def _bn_body(s_ref, st_ref, g_ref, b_ref, o_ref):
    st = st_ref[...]
    s1 = jnp.sum(st[:NW], axis=0, keepdims=True)
    s2 = jnp.sum(st[NW:], axis=0, keepdims=True)
    mean = s1 / RR
    var = s2 / RR - mean * mean
    alpha = g_ref[...] * lax.rsqrt(var + (KK * KK) * EPS)
    shift = b_ref[...] - mean * alpha
    o_ref[...] = s_ref[...] * alpha + shift


def _bn(sums, stats2d, gamma2d, beta2d):
    return pl.pallas_call(
        _bn_body,
        grid=(MM_GRID,),
        in_specs=[
            pl.BlockSpec((MM_BLK, CO), lambda i: (i, 0)),
            pl.BlockSpec((2 * NW, CO), lambda i: (0, 0)),
            pl.BlockSpec((1, CO), lambda i: (0, 0)),
            pl.BlockSpec((1, CO), lambda i: (0, 0)),
        ],
        out_specs=pl.BlockSpec((MM_BLK, CO), lambda i: (i, 0)),
        out_shape=jax.ShapeDtypeStruct((RR, CO), jnp.float32),
    )(sums, stats2d, gamma2d, beta2d)


@jax.jit
def kernel(x, knn, W, gamma, beta):
    h = _matmul(x.reshape(RR, CI), W)
    sums, stats = _make_gather_mean()(h, knn.reshape(RR * KK))
    out = _bn(sums, stats.reshape(2 * NW, CO),
              gamma.reshape(1, CO), beta.reshape(1, CO))
    return out.reshape(NB, NN, CO)
